# baseline (speedup 1.0000x reference)
import functools

import jax
import jax.numpy as jnp
from jax import lax
from jax.experimental import pallas as pl
from jax.experimental.pallas import tpu as pltpu
from jax.experimental.pallas import tpu_sc as plsc

_NUM_ALT = 4
_D = 256
_B = 1024
_L = 200
_BB = 64
_NBLK = _B // _BB
_H = 4
_VMEM_LIMIT = 112 * 1024 * 1024


def _sc_gather(gamma, beta, alt_idx):
    info = plsc.get_sparse_core_info()
    nc, ns = info.num_cores, info.num_subcores
    nw = nc * ns
    b_per_w = _B // nw

    mesh = plsc.VectorSubcoreMesh(core_axis_name="c", subcore_axis_name="s")

    @functools.partial(
        pl.kernel,
        mesh=mesh,
        out_type=[
            jax.ShapeDtypeStruct((_B, _D), jnp.float32),
            jax.ShapeDtypeStruct((_B, _D), jnp.float32),
        ],
        scratch_types=[
            pltpu.VMEM((b_per_w,), jnp.int32),
            pltpu.VMEM((b_per_w, _D), jnp.float32),
            pltpu.VMEM((b_per_w, _D), jnp.float32),
            pltpu.SemaphoreType.DMA,
            pltpu.SemaphoreType.DMA,
        ],
    )
    def gather_kernel(gamma_hbm, beta_hbm, idx_hbm, g_out, b_out,
                      idx_v, grow_v, brow_v, sem_g, sem_b):
        wid = lax.axis_index("s") * nc + lax.axis_index("c")
        base = wid * b_per_w
        pltpu.sync_copy(idx_hbm.at[pl.ds(base, b_per_w)], idx_v)
        cp_g = pltpu.async_copy(gamma_hbm.at[idx_v], grow_v, sem_g)
        cp_b = pltpu.async_copy(beta_hbm.at[idx_v], brow_v, sem_b)
        cp_g.wait()
        cp_b.wait()
        pltpu.sync_copy(grow_v, g_out.at[pl.ds(base, b_per_w)])
        pltpu.sync_copy(brow_v, b_out.at[pl.ds(base, b_per_w)])

    return gather_kernel(gamma, beta, alt_idx)


def _select_body(idx_ref, gamma_ref, beta_ref, feat_ref, out_ref):
    idx = idx_ref[...]
    bb = idx.shape[0]
    g = jnp.broadcast_to(gamma_ref[0, :][None, :], (bb, _D))
    b = jnp.broadcast_to(beta_ref[0, :][None, :], (bb, _D))
    for k in range(1, _NUM_ALT):
        sel = idx == k
        g = jnp.where(sel, gamma_ref[k, :][None, :], g)
        b = jnp.where(sel, beta_ref[k, :][None, :], b)
    out_ref[...] = feat_ref[...] * g[:, None, :] + b[:, None, :]


def _rows_body(g_ref, b_ref, feat_ref, prev_ref, out_ref):
    del prev_ref
    g = g_ref[...][:, None, :]
    b = b_ref[...][:, None, :]
    out_ref[...] = feat_ref[...] * g + b


def _tc_select_part(feat, alt_idx, gamma, beta):
    idx2 = alt_idx.astype(jnp.int32).reshape(_B, 1)
    return pl.pallas_call(
        _select_body,
        grid=(_H,),
        in_specs=[
            pl.BlockSpec((_BB, 1), lambda i: (i, 0)),
            pl.BlockSpec((_NUM_ALT, _D), lambda i: (0, 0)),
            pl.BlockSpec((_NUM_ALT, _D), lambda i: (0, 0)),
            pl.BlockSpec((_BB, _L, _D), lambda i: (i, 0, 0)),
        ],
        out_specs=pl.BlockSpec((_BB, _L, _D), lambda i: (i, 0, 0)),
        out_shape=jax.ShapeDtypeStruct((_B, _L, _D), jnp.float32),
        compiler_params=pltpu.CompilerParams(
            dimension_semantics=("arbitrary",),
            vmem_limit_bytes=_VMEM_LIMIT,
        ),
    )(idx2, gamma, beta, feat)


def _tc_rows_part(feat, g, b, prev):
    return pl.pallas_call(
        _rows_body,
        grid=(_NBLK - _H,),
        in_specs=[
            pl.BlockSpec((_BB, _D), lambda i: (i + _H, 0)),
            pl.BlockSpec((_BB, _D), lambda i: (i + _H, 0)),
            pl.BlockSpec((_BB, _L, _D), lambda i: (i + _H, 0, 0)),
            pl.BlockSpec(memory_space=pl.ANY),
        ],
        out_specs=pl.BlockSpec((_BB, _L, _D), lambda i: (i + _H, 0, 0)),
        out_shape=jax.ShapeDtypeStruct((_B, _L, _D), jnp.float32),
        input_output_aliases={3: 0},
        compiler_params=pltpu.CompilerParams(
            dimension_semantics=("arbitrary",),
            vmem_limit_bytes=_VMEM_LIMIT,
        ),
    )(g, b, feat, prev)


def kernel(feat, alt_idx, gamma, beta):
    g, b = _sc_gather(gamma, beta, alt_idx.astype(jnp.int32))
    partial = _tc_select_part(feat, alt_idx, gamma, beta)
    return _tc_rows_part(feat, g, b, partial)


# speedup vs baseline: 1.2197x; 1.2197x over previous
import jax
import jax.numpy as jnp
from jax.experimental import pallas as pl
from jax.experimental.pallas import tpu as pltpu

_NUM_ALT = 4
_D = 256
_B = 1024
_L = 200
_BB = 64


def _fused_body(idx_ref, gamma_ref, beta_ref, feat_ref, out_ref):
    idx = idx_ref[...]
    g = jnp.broadcast_to(gamma_ref[0, :][None, :], (_BB, _D))
    b = jnp.broadcast_to(beta_ref[0, :][None, :], (_BB, _D))
    for k in range(1, _NUM_ALT):
        sel = idx == k
        g = jnp.where(sel, gamma_ref[k, :][None, :], g)
        b = jnp.where(sel, beta_ref[k, :][None, :], b)
    out_ref[...] = feat_ref[...] * g[:, None, :] + b[:, None, :]


def kernel(feat, alt_idx, gamma, beta):
    idx2 = alt_idx.astype(jnp.int32).reshape(_B, 1)
    return pl.pallas_call(
        _fused_body,
        grid=(_B // _BB,),
        in_specs=[
            pl.BlockSpec((_BB, 1), lambda i: (i, 0)),
            pl.BlockSpec((_NUM_ALT, _D), lambda i: (0, 0)),
            pl.BlockSpec((_NUM_ALT, _D), lambda i: (0, 0)),
            pl.BlockSpec((_BB, _L, _D), lambda i: (i, 0, 0)),
        ],
        out_specs=pl.BlockSpec((_BB, _L, _D), lambda i: (i, 0, 0)),
        out_shape=jax.ShapeDtypeStruct((_B, _L, _D), jnp.float32),
        compiler_params=pltpu.CompilerParams(
            dimension_semantics=("arbitrary",),
            vmem_limit_bytes=112 * 1024 * 1024,
        ),
    )(idx2, gamma, beta, feat)
